# Initial kernel scaffold; baseline (speedup 1.0000x reference)
import jax
import jax.numpy as jnp
from jax.experimental import pallas as pl


def kernel(u0, edge_index, ts, W_enc, b_enc, We, be, Wn, bn, W_dec, b_dec):
    raise NotImplementedError("write your pallas kernel here")


# speedup vs baseline: 3.7729x; 3.7729x over previous
import functools

import jax
import jax.numpy as jnp
from jax import lax
from jax.experimental import pallas as pl
from jax.experimental.pallas import tpu as pltpu
from jax.experimental.pallas import tpu_sc as plsc

N_NODES = 50000
N_EDGES = 800000
D_IN = 128
D_LAT = 64
HALF = D_LAT // 2
N_LAYERS = 4
T_STEPS = 3

RB = 5000
GRID = N_NODES // RB

NS = 16
EPT = N_EDGES // NS
CH = 128
NFULL = EPT // CH
REM = EPT - NFULL * CH
NROWF = N_NODES // CH
ROWREM = N_NODES - NROWF * CH


def _enc_body(u_ref, w_ref, b_ref, h_ref):
    h_ref[...] = jnp.maximum(
        jnp.dot(u_ref[...], w_ref[...], preferred_element_type=jnp.float32)
        + b_ref[...], 0.0)


def _encode(u, W_enc, b_enc2):
    return pl.pallas_call(
        _enc_body,
        grid=(GRID,),
        in_specs=[
            pl.BlockSpec((RB, D_IN), lambda i: (i, 0)),
            pl.BlockSpec((D_IN, D_LAT), lambda i: (0, 0)),
            pl.BlockSpec((1, D_LAT), lambda i: (0, 0)),
        ],
        out_specs=pl.BlockSpec((RB, D_LAT), lambda i: (i, 0)),
        out_shape=jax.ShapeDtypeStruct((N_NODES, D_LAT), jnp.float32),
    )(u, W_enc, b_enc2)


def _proj_body(h_ref, wt_ref, wb_ref, be_ref, a0_ref, a1_ref, b0_ref, b1_ref):
    h = h_ref[...]
    A = jnp.dot(h, wt_ref[...], preferred_element_type=jnp.float32)
    B = jnp.dot(h, wb_ref[...], preferred_element_type=jnp.float32) + be_ref[...]
    a0_ref[...] = A[:, :HALF]
    a1_ref[...] = A[:, HALF:]
    b0_ref[...] = B[:, :HALF]
    b1_ref[...] = B[:, HALF:]


def _project(h, We_t, We_b, be2):
    half = jax.ShapeDtypeStruct((N_NODES, HALF), jnp.float32)
    return pl.pallas_call(
        _proj_body,
        grid=(GRID,),
        in_specs=[
            pl.BlockSpec((RB, D_LAT), lambda i: (i, 0)),
            pl.BlockSpec((D_LAT, D_LAT), lambda i: (0, 0)),
            pl.BlockSpec((D_LAT, D_LAT), lambda i: (0, 0)),
            pl.BlockSpec((1, D_LAT), lambda i: (0, 0)),
        ],
        out_specs=[pl.BlockSpec((RB, HALF), lambda i: (i, 0))] * 4,
        out_shape=[half, half, half, half],
    )(h, We_t, We_b, be2)


def _node_body(h_ref, g0_ref, g1_ref, w_ref, b_ref, o_ref):
    h = h_ref[...]
    w = w_ref[...]
    acc = jnp.dot(h, w[:D_LAT], preferred_element_type=jnp.float32)
    acc = acc + jnp.dot(g0_ref[...], w[D_LAT:D_LAT + HALF],
                        preferred_element_type=jnp.float32)
    acc = acc + jnp.dot(g1_ref[...], w[D_LAT + HALF:],
                        preferred_element_type=jnp.float32)
    o_ref[...] = h + jnp.maximum(acc + b_ref[...], 0.0)


def _node_update(h, g0, g1, Wn_l, bn2):
    return pl.pallas_call(
        _node_body,
        grid=(GRID,),
        in_specs=[
            pl.BlockSpec((RB, D_LAT), lambda i: (i, 0)),
            pl.BlockSpec((RB, HALF), lambda i: (i, 0)),
            pl.BlockSpec((RB, HALF), lambda i: (i, 0)),
            pl.BlockSpec((2 * D_LAT, D_LAT), lambda i: (0, 0)),
            pl.BlockSpec((1, D_LAT), lambda i: (0, 0)),
        ],
        out_specs=pl.BlockSpec((RB, D_LAT), lambda i: (i, 0)),
        out_shape=jax.ShapeDtypeStruct((N_NODES, D_LAT), jnp.float32),
    )(h, g0, g1, Wn_l, bn2)


def _dec_body(h_ref, y_ref, w_ref, b_ref, dt_ref, o_ref):
    dy = jnp.dot(h_ref[...], w_ref[...],
                 preferred_element_type=jnp.float32) + b_ref[...]
    o_ref[...] = y_ref[...] + dt_ref[...] * dy


def _decode(h, y, W_dec, b_dec2, dt):
    return pl.pallas_call(
        _dec_body,
        grid=(GRID,),
        in_specs=[
            pl.BlockSpec((RB, D_LAT), lambda i: (i, 0)),
            pl.BlockSpec((RB, D_IN), lambda i: (i, 0)),
            pl.BlockSpec((D_LAT, D_IN), lambda i: (0, 0)),
            pl.BlockSpec((1, D_IN), lambda i: (0, 0)),
            pl.BlockSpec((1, 1), lambda i: (0, 0)),
        ],
        out_specs=pl.BlockSpec((RB, D_IN), lambda i: (i, 0)),
        out_shape=jax.ShapeDtypeStruct((N_NODES, D_IN), jnp.float32),
    )(h, y, W_dec, b_dec2, dt)


def _mp_body(a0_hbm, a1_hbm, b0_hbm, b1_hbm, src_hbm, dst_hbm,
             out0_hbm, out1_hbm,
             idx_s, idx_d, bufA, bufB, agg_sh, sem):
    c = lax.axis_index("c")
    s = lax.axis_index("s")

    zero16 = jnp.zeros((16,), jnp.float32)

    def zrow(e, _):
        bufA[e, 0:16] = zero16
        bufA[e, 16:32] = zero16
        return 0

    lax.fori_loop(0, CH, zrow, 0)

    nz = (NROWF + NS - 1) // NS

    def zchunk(k, _):
        ci = s + NS * k

        @pl.when(ci < NROWF)
        def _():
            pltpu.sync_copy(bufA, agg_sh.at[pl.ds(ci * CH, CH)])
        return 0

    lax.fori_loop(0, nz, zchunk, 0)

    @pl.when(s == 0)
    def _():
        pltpu.sync_copy(bufA.at[pl.ds(0, ROWREM)],
                        agg_sh.at[pl.ds(NROWF * CH, ROWREM)])

    plsc.subcore_barrier()

    ebase = s * EPT

    def do_chunk(off, n):
        pltpu.sync_copy(src_hbm.at[pl.ds(off, n)], idx_s.at[pl.ds(0, n)])
        pltpu.sync_copy(dst_hbm.at[pl.ds(off, n)], idx_d.at[pl.ds(0, n)])

        @pl.when(c == 0)
        def _():
            pltpu.async_copy(a0_hbm.at[idx_s], bufA, sem).wait()
            pltpu.async_copy(b0_hbm.at[idx_d], bufB, sem).wait()

        @pl.when(c == 1)
        def _():
            pltpu.async_copy(a1_hbm.at[idx_s], bufA, sem).wait()
            pltpu.async_copy(b1_hbm.at[idx_d], bufB, sem).wait()

        def ew(e, _):
            bufA[e, 0:16] = jnp.maximum(bufA[e, 0:16] + bufB[e, 0:16], 0.0)
            bufA[e, 16:32] = jnp.maximum(bufA[e, 16:32] + bufB[e, 16:32], 0.0)
            return 0

        lax.fori_loop(0, n, ew, 0)
        if n < CH:
            lax.fori_loop(n, CH, zrow, 0)
        pltpu.sync_copy(bufA, agg_sh.at[idx_d], add=True)

    def chunk_body(i, _):
        do_chunk(ebase + i * CH, CH)
        return 0

    lax.fori_loop(0, NFULL, chunk_body, 0)
    do_chunk(ebase + NFULL * CH, REM)

    plsc.subcore_barrier()

    def copy_rows(r0, n):
        pltpu.sync_copy(agg_sh.at[pl.ds(r0, n)], bufA.at[pl.ds(0, n)])

        @pl.when(c == 0)
        def _():
            pltpu.sync_copy(bufA.at[pl.ds(0, n)], out0_hbm.at[pl.ds(r0, n)])

        @pl.when(c == 1)
        def _():
            pltpu.sync_copy(bufA.at[pl.ds(0, n)], out1_hbm.at[pl.ds(r0, n)])

    def ochunk(k, _):
        ci = s + NS * k

        @pl.when(ci < NROWF)
        def _():
            copy_rows(ci * CH, CH)
        return 0

    lax.fori_loop(0, nz, ochunk, 0)

    @pl.when(s == 0)
    def _():
        copy_rows(NROWF * CH, ROWREM)


def _message(a0, a1, b0, b1, src, dst):
    mesh = plsc.VectorSubcoreMesh(core_axis_name="c", subcore_axis_name="s")
    half = jax.ShapeDtypeStruct((N_NODES, HALF), jnp.float32)
    f = pl.kernel(
        _mp_body,
        out_type=[half, half],
        mesh=mesh,
        compiler_params=pltpu.CompilerParams(use_tc_tiling_on_sc=False),
        scratch_types=[
            pltpu.VMEM((CH,), jnp.int32),
            pltpu.VMEM((CH,), jnp.int32),
            pltpu.VMEM((CH, HALF), jnp.float32),
            pltpu.VMEM((CH, HALF), jnp.float32),
            pltpu.VMEM_SHARED((N_NODES, HALF), jnp.float32),
            pltpu.SemaphoreType.DMA,
        ],
    )
    return f(a0, a1, b0, b1, src, dst)


def kernel(u0, edge_index, ts, W_enc, b_enc, We, be, Wn, bn, W_dec, b_dec):
    ei = edge_index.astype(jnp.int32)
    src = ei[0]
    dst = ei[1]
    b_enc2 = b_enc.reshape(1, D_LAT)
    b_dec2 = b_dec.reshape(1, D_IN)

    y = u0
    ys = [y]
    for k in range(T_STEPS - 1):
        dt = (ts[k + 1] - ts[k]).reshape(1, 1)
        h = _encode(y, W_enc, b_enc2)
        for l in range(N_LAYERS):
            a0, a1, bb0, bb1 = _project(h, We[l][:D_LAT], We[l][D_LAT:],
                                        be[l].reshape(1, D_LAT))
            g0, g1 = _message(a0, a1, bb0, bb1, src, dst)
            h = _node_update(h, g0, g1, Wn[l], bn[l].reshape(1, D_LAT))
        y = _decode(h, y, W_dec, b_dec2, dt)
        ys.append(y)
    return jnp.stack(ys, axis=0)


# speedup vs baseline: 5.6664x; 1.5019x over previous
import functools

import jax
import jax.numpy as jnp
from jax import lax
from jax.experimental import pallas as pl
from jax.experimental.pallas import tpu as pltpu
from jax.experimental.pallas import tpu_sc as plsc

N_NODES = 50000
N_EDGES = 800000
D_IN = 128
D_LAT = 64
HALF = D_LAT // 2
N_LAYERS = 4
T_STEPS = 3

RB = 5000
GRID = N_NODES // RB

NS = 16
CH = 128
CPB = 8
BPT = 50
CPT = BPT * CPB
PAD_E = NS * CPT * CH
IDX_ROWS = PAD_E // CH
NROWF = N_NODES // CH
ROWREM = N_NODES - NROWF * CH


def _enc_body(u_ref, w_ref, b_ref, h_ref):
    h_ref[...] = jnp.maximum(
        jnp.dot(u_ref[...], w_ref[...], preferred_element_type=jnp.float32)
        + b_ref[...], 0.0)


def _encode(u, W_enc, b_enc2):
    return pl.pallas_call(
        _enc_body,
        grid=(GRID,),
        in_specs=[
            pl.BlockSpec((RB, D_IN), lambda i: (i, 0)),
            pl.BlockSpec((D_IN, D_LAT), lambda i: (0, 0)),
            pl.BlockSpec((1, D_LAT), lambda i: (0, 0)),
        ],
        out_specs=pl.BlockSpec((RB, D_LAT), lambda i: (i, 0)),
        out_shape=jax.ShapeDtypeStruct((N_NODES, D_LAT), jnp.float32),
    )(u, W_enc, b_enc2)


def _proj_body(h_ref, wt_ref, wb_ref, be_ref, a0_ref, a1_ref, b0_ref, b1_ref):
    h = h_ref[...]
    A = jnp.dot(h, wt_ref[...], preferred_element_type=jnp.float32)
    B = jnp.dot(h, wb_ref[...], preferred_element_type=jnp.float32) + be_ref[...]
    a0_ref[...] = A[:, :HALF]
    a1_ref[...] = A[:, HALF:]
    b0_ref[...] = B[:, :HALF]
    b1_ref[...] = B[:, HALF:]


def _project(h, We_t, We_b, be2):
    half = jax.ShapeDtypeStruct((N_NODES, HALF), jnp.float32)
    return pl.pallas_call(
        _proj_body,
        grid=(GRID,),
        in_specs=[
            pl.BlockSpec((RB, D_LAT), lambda i: (i, 0)),
            pl.BlockSpec((D_LAT, D_LAT), lambda i: (0, 0)),
            pl.BlockSpec((D_LAT, D_LAT), lambda i: (0, 0)),
            pl.BlockSpec((1, D_LAT), lambda i: (0, 0)),
        ],
        out_specs=[pl.BlockSpec((RB, HALF), lambda i: (i, 0))] * 4,
        out_shape=[half, half, half, half],
    )(h, We_t, We_b, be2)


def _node_body(h_ref, g0_ref, g1_ref, w_ref, b_ref, o_ref):
    h = h_ref[...]
    w = w_ref[...]
    acc = jnp.dot(h, w[:D_LAT], preferred_element_type=jnp.float32)
    acc = acc + jnp.dot(g0_ref[...], w[D_LAT:D_LAT + HALF],
                        preferred_element_type=jnp.float32)
    acc = acc + jnp.dot(g1_ref[...], w[D_LAT + HALF:],
                        preferred_element_type=jnp.float32)
    o_ref[...] = h + jnp.maximum(acc + b_ref[...], 0.0)


def _node_update(h, g0, g1, Wn_l, bn2):
    return pl.pallas_call(
        _node_body,
        grid=(GRID,),
        in_specs=[
            pl.BlockSpec((RB, D_LAT), lambda i: (i, 0)),
            pl.BlockSpec((RB, HALF), lambda i: (i, 0)),
            pl.BlockSpec((RB, HALF), lambda i: (i, 0)),
            pl.BlockSpec((2 * D_LAT, D_LAT), lambda i: (0, 0)),
            pl.BlockSpec((1, D_LAT), lambda i: (0, 0)),
        ],
        out_specs=pl.BlockSpec((RB, D_LAT), lambda i: (i, 0)),
        out_shape=jax.ShapeDtypeStruct((N_NODES, D_LAT), jnp.float32),
    )(h, g0, g1, Wn_l, bn2)


def _dec_body(h_ref, y_ref, w_ref, b_ref, dt_ref, o_ref):
    dy = jnp.dot(h_ref[...], w_ref[...],
                 preferred_element_type=jnp.float32) + b_ref[...]
    o_ref[...] = y_ref[...] + dt_ref[...] * dy


def _decode(h, y, W_dec, b_dec2, dt):
    return pl.pallas_call(
        _dec_body,
        grid=(GRID,),
        in_specs=[
            pl.BlockSpec((RB, D_LAT), lambda i: (i, 0)),
            pl.BlockSpec((RB, D_IN), lambda i: (i, 0)),
            pl.BlockSpec((D_LAT, D_IN), lambda i: (0, 0)),
            pl.BlockSpec((1, D_IN), lambda i: (0, 0)),
            pl.BlockSpec((1, 1), lambda i: (0, 0)),
        ],
        out_specs=pl.BlockSpec((RB, D_IN), lambda i: (i, 0)),
        out_shape=jax.ShapeDtypeStruct((N_NODES, D_IN), jnp.float32),
    )(h, y, W_dec, b_dec2, dt)


def _mp_body(a0_hbm, a1_hbm, b0_hbm, b1_hbm, src_hbm, dst_hbm,
             out0_hbm, out1_hbm,
             is0, is1, id0, id1, bufA0, bufB0, bufA1, bufB1,
             agg_sh, isem, gsem0, gsem1):
    c = lax.axis_index("c")
    s = lax.axis_index("s")

    ISL = (is0, is1)
    IDL = (id0, id1)
    BA = (bufA0, bufA1)
    BB = (bufB0, bufB1)
    GS = (gsem0, gsem1)

    zero16 = jnp.zeros((16,), jnp.float32)

    def zero_rows(ref, lo, hi):
        def zr(e, _):
            ref[e, 0:16] = zero16
            ref[e, 16:32] = zero16
            return 0
        lax.fori_loop(lo, hi, zr, 0)

    zero_rows(bufA0, 0, CH)
    nz = (NROWF + NS - 1) // NS

    def zchunk(k, _):
        ci = s + NS * k

        @pl.when(ci < NROWF)
        def _():
            pltpu.sync_copy(bufA0, agg_sh.at[pl.ds(ci * CH, CH)])
        return 0

    lax.fori_loop(0, nz, zchunk, 0)

    @pl.when(s == 0)
    def _():
        pltpu.sync_copy(bufA0.at[pl.ds(0, ROWREM)],
                        agg_sh.at[pl.ds(NROWF * CH, ROWREM)])

    plsc.subcore_barrier()

    rowbase = s * CPT

    def pipeline(a_hbm, b_hbm, out_hbm):
        def issue(p, isl, idl, j):
            pltpu.async_copy(a_hbm.at[isl.at[j]], BA[p], GS[p])
            pltpu.async_copy(b_hbm.at[idl.at[j]], BB[p], GS[p])

        def process(p, idl, j, off):
            pltpu.make_async_copy(a_hbm.at[pl.ds(0, CH)], BA[p], GS[p]).wait()
            pltpu.make_async_copy(b_hbm.at[pl.ds(0, CH)], BB[p], GS[p]).wait()
            bA = BA[p]
            bB = BB[p]

            def ew(e, _):
                bA[e, 0:16] = jnp.maximum(bA[e, 0:16] + bB[e, 0:16], 0.0)
                bA[e, 16:32] = jnp.maximum(bA[e, 16:32] + bB[e, 16:32], 0.0)
                return 0

            lax.fori_loop(0, CH, ew, 0, unroll=4)
            valid = jnp.minimum(CH, jnp.maximum(0, N_EDGES - off))
            zero_rows(bA, valid, CH)
            pltpu.sync_copy(bA, agg_sh.at[idl.at[j]], add=True)

        pltpu.sync_copy(src_hbm.at[pl.ds(rowbase, CPB)], is0)
        pltpu.sync_copy(dst_hbm.at[pl.ds(rowbase, CPB)], id0)
        issue(0, is0, id0, 0)

        def blockpair(b2, _):
            for half_i in range(2):
                bb = 2 * b2 + half_i
                isl, idl = ISL[half_i], IDL[half_i]
                isl2, idl2 = ISL[1 - half_i], IDL[1 - half_i]
                rb = rowbase + bb * CPB

                @pl.when(bb + 1 < BPT)
                def _():
                    pltpu.async_copy(src_hbm.at[pl.ds(rb + CPB, CPB)],
                                     isl2, isem)
                    pltpu.async_copy(dst_hbm.at[pl.ds(rb + CPB, CPB)],
                                     idl2, isem)

                for j in range(CPB):
                    p = j % 2
                    if j < CPB - 1:
                        issue(1 - p, isl, idl, j + 1)
                    else:
                        @pl.when(bb + 1 < BPT)
                        def _():
                            pltpu.make_async_copy(
                                src_hbm.at[pl.ds(0, CPB)], isl2, isem).wait()
                            pltpu.make_async_copy(
                                dst_hbm.at[pl.ds(0, CPB)], idl2, isem).wait()
                            issue(1 - p, isl2, idl2, 0)
                    process(p, idl, j, (rb + j) * CH)
            return 0

        lax.fori_loop(0, BPT // 2, blockpair, 0)

        plsc.subcore_barrier()

        def ochunk(k, _):
            ci = s + NS * k

            @pl.when(ci < NROWF)
            def _():
                pltpu.sync_copy(agg_sh.at[pl.ds(ci * CH, CH)], bufA0)
                pltpu.sync_copy(bufA0, out_hbm.at[pl.ds(ci * CH, CH)])
            return 0

        lax.fori_loop(0, nz, ochunk, 0)

        @pl.when(s == 0)
        def _():
            pltpu.sync_copy(agg_sh.at[pl.ds(NROWF * CH, ROWREM)],
                            bufA0.at[pl.ds(0, ROWREM)])
            pltpu.sync_copy(bufA0.at[pl.ds(0, ROWREM)],
                            out_hbm.at[pl.ds(NROWF * CH, ROWREM)])

    @pl.when(c == 0)
    def _():
        pipeline(a0_hbm, b0_hbm, out0_hbm)

    @pl.when(c == 1)
    def _():
        pipeline(a1_hbm, b1_hbm, out1_hbm)


def _message(a0, a1, b0, b1, srcp, dstp):
    mesh = plsc.VectorSubcoreMesh(core_axis_name="c", subcore_axis_name="s")
    half = jax.ShapeDtypeStruct((N_NODES, HALF), jnp.float32)
    f = pl.kernel(
        _mp_body,
        out_type=[half, half],
        mesh=mesh,
        compiler_params=pltpu.CompilerParams(use_tc_tiling_on_sc=False),
        scratch_types=[
            pltpu.VMEM((CPB, CH), jnp.int32),
            pltpu.VMEM((CPB, CH), jnp.int32),
            pltpu.VMEM((CPB, CH), jnp.int32),
            pltpu.VMEM((CPB, CH), jnp.int32),
            pltpu.VMEM((CH, HALF), jnp.float32),
            pltpu.VMEM((CH, HALF), jnp.float32),
            pltpu.VMEM((CH, HALF), jnp.float32),
            pltpu.VMEM((CH, HALF), jnp.float32),
            pltpu.VMEM_SHARED((N_NODES, HALF), jnp.float32),
            pltpu.SemaphoreType.DMA,
            pltpu.SemaphoreType.DMA,
            pltpu.SemaphoreType.DMA,
        ],
    )
    return f(a0, a1, b0, b1, srcp, dstp)


def kernel(u0, edge_index, ts, W_enc, b_enc, We, be, Wn, bn, W_dec, b_dec):
    ei = edge_index.astype(jnp.int32)
    pad = jnp.zeros((PAD_E - N_EDGES,), jnp.int32)
    srcp = jnp.concatenate([ei[0], pad]).reshape(IDX_ROWS, CH)
    dstp = jnp.concatenate([ei[1], pad]).reshape(IDX_ROWS, CH)
    b_enc2 = b_enc.reshape(1, D_LAT)
    b_dec2 = b_dec.reshape(1, D_IN)

    y = u0
    ys = [y]
    for k in range(T_STEPS - 1):
        dt = (ts[k + 1] - ts[k]).reshape(1, 1)
        h = _encode(y, W_enc, b_enc2)
        for l in range(N_LAYERS):
            a0, a1, bb0, bb1 = _project(h, We[l][:D_LAT], We[l][D_LAT:],
                                        be[l].reshape(1, D_LAT))
            g0, g1 = _message(a0, a1, bb0, bb1, srcp, dstp)
            h = _node_update(h, g0, g1, Wn[l], bn[l].reshape(1, D_LAT))
        y = _decode(h, y, W_dec, b_dec2, dt)
        ys.append(y)
    return jnp.stack(ys, axis=0)


# speedup vs baseline: 5.8038x; 1.0242x over previous
import functools

import jax
import jax.numpy as jnp
from jax import lax
from jax.experimental import pallas as pl
from jax.experimental.pallas import tpu as pltpu
from jax.experimental.pallas import tpu_sc as plsc

N_NODES = 50000
N_EDGES = 800000
D_IN = 128
D_LAT = 64
HALF = D_LAT // 2
N_LAYERS = 4
T_STEPS = 3

RB = 5000
GRID = N_NODES // RB

NS = 16
CH = 128
CPB = 8
BPT = 50
CPT = BPT * CPB
PAD_E = NS * CPT * CH
IDX_ROWS = PAD_E // CH
NROWF = N_NODES // CH
ROWREM = N_NODES - NROWF * CH


def _enc_body(u_ref, w_ref, b_ref, h_ref):
    h_ref[...] = jnp.maximum(
        jnp.dot(u_ref[...], w_ref[...], preferred_element_type=jnp.float32)
        + b_ref[...], 0.0)


def _encode(u, W_enc, b_enc2):
    return pl.pallas_call(
        _enc_body,
        grid=(GRID,),
        in_specs=[
            pl.BlockSpec((RB, D_IN), lambda i: (i, 0)),
            pl.BlockSpec((D_IN, D_LAT), lambda i: (0, 0)),
            pl.BlockSpec((1, D_LAT), lambda i: (0, 0)),
        ],
        out_specs=pl.BlockSpec((RB, D_LAT), lambda i: (i, 0)),
        out_shape=jax.ShapeDtypeStruct((N_NODES, D_LAT), jnp.float32),
    )(u, W_enc, b_enc2)


def _proj_body(h_ref, wt_ref, wb_ref, be_ref, a0_ref, a1_ref, b0_ref, b1_ref):
    h = h_ref[...]
    A = jnp.dot(h, wt_ref[...], preferred_element_type=jnp.float32)
    B = jnp.dot(h, wb_ref[...], preferred_element_type=jnp.float32) + be_ref[...]
    a0_ref[...] = A[:, :HALF]
    a1_ref[...] = A[:, HALF:]
    b0_ref[...] = B[:, :HALF]
    b1_ref[...] = B[:, HALF:]


def _project(h, We_t, We_b, be2):
    half = jax.ShapeDtypeStruct((N_NODES, HALF), jnp.float32)
    return pl.pallas_call(
        _proj_body,
        grid=(GRID,),
        in_specs=[
            pl.BlockSpec((RB, D_LAT), lambda i: (i, 0)),
            pl.BlockSpec((D_LAT, D_LAT), lambda i: (0, 0)),
            pl.BlockSpec((D_LAT, D_LAT), lambda i: (0, 0)),
            pl.BlockSpec((1, D_LAT), lambda i: (0, 0)),
        ],
        out_specs=[pl.BlockSpec((RB, HALF), lambda i: (i, 0))] * 4,
        out_shape=[half, half, half, half],
    )(h, We_t, We_b, be2)


def _node_body(h_ref, g0_ref, g1_ref, w_ref, b_ref, o_ref):
    h = h_ref[...]
    w = w_ref[...]
    acc = jnp.dot(h, w[:D_LAT], preferred_element_type=jnp.float32)
    acc = acc + jnp.dot(g0_ref[...], w[D_LAT:D_LAT + HALF],
                        preferred_element_type=jnp.float32)
    acc = acc + jnp.dot(g1_ref[...], w[D_LAT + HALF:],
                        preferred_element_type=jnp.float32)
    o_ref[...] = h + jnp.maximum(acc + b_ref[...], 0.0)


def _node_update(h, g0, g1, Wn_l, bn2):
    return pl.pallas_call(
        _node_body,
        grid=(GRID,),
        in_specs=[
            pl.BlockSpec((RB, D_LAT), lambda i: (i, 0)),
            pl.BlockSpec((RB, HALF), lambda i: (i, 0)),
            pl.BlockSpec((RB, HALF), lambda i: (i, 0)),
            pl.BlockSpec((2 * D_LAT, D_LAT), lambda i: (0, 0)),
            pl.BlockSpec((1, D_LAT), lambda i: (0, 0)),
        ],
        out_specs=pl.BlockSpec((RB, D_LAT), lambda i: (i, 0)),
        out_shape=jax.ShapeDtypeStruct((N_NODES, D_LAT), jnp.float32),
    )(h, g0, g1, Wn_l, bn2)


def _dec_body(h_ref, y_ref, w_ref, b_ref, dt_ref, o_ref):
    dy = jnp.dot(h_ref[...], w_ref[...],
                 preferred_element_type=jnp.float32) + b_ref[...]
    o_ref[...] = y_ref[...] + dt_ref[...] * dy


def _decode(h, y, W_dec, b_dec2, dt):
    return pl.pallas_call(
        _dec_body,
        grid=(GRID,),
        in_specs=[
            pl.BlockSpec((RB, D_LAT), lambda i: (i, 0)),
            pl.BlockSpec((RB, D_IN), lambda i: (i, 0)),
            pl.BlockSpec((D_LAT, D_IN), lambda i: (0, 0)),
            pl.BlockSpec((1, D_IN), lambda i: (0, 0)),
            pl.BlockSpec((1, 1), lambda i: (0, 0)),
        ],
        out_specs=pl.BlockSpec((RB, D_IN), lambda i: (i, 0)),
        out_shape=jax.ShapeDtypeStruct((N_NODES, D_IN), jnp.float32),
    )(h, y, W_dec, b_dec2, dt)


def _mp_body(a0_hbm, a1_hbm, b0_hbm, b1_hbm, src_hbm, dst_hbm,
             out0_hbm, out1_hbm,
             is0, is1, id0, id1, bufA0, bufB0, bufA1, bufB1, m0, m1,
             agg_sh, isem, gsem0, gsem1, ssem0, ssem1):
    c = lax.axis_index("c")
    s = lax.axis_index("s")

    ISL = (is0, is1)
    IDL = (id0, id1)
    BA = (bufA0, bufA1)
    BB = (bufB0, bufB1)
    MM = (m0, m1)
    GS = (gsem0, gsem1)
    SS = (ssem0, ssem1)

    zero16 = jnp.zeros((16,), jnp.float32)

    def zero_rows(ref, lo, hi):
        def zr(e, _):
            ref[e, 0:16] = zero16
            ref[e, 16:32] = zero16
            return 0
        lax.fori_loop(lo, hi, zr, 0)

    zero_rows(bufA0, 0, CH)
    nz = (NROWF + NS - 1) // NS

    def zchunk(k, _):
        ci = s + NS * k

        @pl.when(ci < NROWF)
        def _():
            pltpu.sync_copy(bufA0, agg_sh.at[pl.ds(ci * CH, CH)])
        return 0

    lax.fori_loop(0, nz, zchunk, 0)

    @pl.when(s == 0)
    def _():
        pltpu.sync_copy(bufA0.at[pl.ds(0, ROWREM)],
                        agg_sh.at[pl.ds(NROWF * CH, ROWREM)])

    plsc.subcore_barrier()

    rowbase = s * CPT

    def pipeline(a_hbm, b_hbm, out_hbm):
        def issue(p, isl, idl, j):
            pltpu.async_copy(a_hbm.at[isl.at[j]], BA[p], GS[p])
            pltpu.async_copy(b_hbm.at[idl.at[j]], BB[p], GS[p])

        def drain_scatter(p):
            pltpu.make_async_copy(m0, agg_sh.at[pl.ds(0, CH)], SS[p]).wait()

        def process(p, idl, j, off):
            pltpu.make_async_copy(a_hbm.at[pl.ds(0, CH)], BA[p], GS[p]).wait()
            pltpu.make_async_copy(b_hbm.at[pl.ds(0, CH)], BB[p], GS[p]).wait()
            bA = BA[p]
            bB = BB[p]
            bM = MM[p]

            def ew(e, _):
                bM[e, 0:16] = jnp.maximum(bA[e, 0:16] + bB[e, 0:16], 0.0)
                bM[e, 16:32] = jnp.maximum(bA[e, 16:32] + bB[e, 16:32], 0.0)
                return 0

            lax.fori_loop(0, CH, ew, 0, unroll=4)
            valid = jnp.minimum(CH, jnp.maximum(0, N_EDGES - off))
            zero_rows(bM, valid, CH)
            pltpu.async_copy(bM, agg_sh.at[idl.at[j]], SS[p], add=True)

        pltpu.sync_copy(src_hbm.at[pl.ds(rowbase, CPB)], is0)
        pltpu.sync_copy(dst_hbm.at[pl.ds(rowbase, CPB)], id0)
        issue(0, is0, id0, 0)

        def blockpair(b2, _):
            for half_i in range(2):
                bb = 2 * b2 + half_i
                isl, idl = ISL[half_i], IDL[half_i]
                isl2, idl2 = ISL[1 - half_i], IDL[1 - half_i]
                rb = rowbase + bb * CPB

                for j in range(CPB):
                    p = j % 2
                    if j >= 2:
                        drain_scatter(p)
                    else:
                        @pl.when(bb >= 1)
                        def _():
                            drain_scatter(p)
                    if j == 1:
                        @pl.when(bb + 1 < BPT)
                        def _():
                            pltpu.async_copy(src_hbm.at[pl.ds(rb + CPB, CPB)],
                                             isl2, isem)
                            pltpu.async_copy(dst_hbm.at[pl.ds(rb + CPB, CPB)],
                                             idl2, isem)
                    if j < CPB - 1:
                        issue(1 - p, isl, idl, j + 1)
                    else:
                        @pl.when(bb + 1 < BPT)
                        def _():
                            pltpu.make_async_copy(
                                src_hbm.at[pl.ds(0, CPB)], isl2, isem).wait()
                            pltpu.make_async_copy(
                                dst_hbm.at[pl.ds(0, CPB)], idl2, isem).wait()
                            issue(1 - p, isl2, idl2, 0)
                    process(p, idl, j, (rb + j) * CH)
            return 0

        lax.fori_loop(0, BPT // 2, blockpair, 0)
        drain_scatter(0)
        drain_scatter(1)

        plsc.subcore_barrier()

        def ochunk(k, _):
            ci = s + NS * k

            @pl.when(ci < NROWF)
            def _():
                pltpu.sync_copy(agg_sh.at[pl.ds(ci * CH, CH)], bufA0)
                pltpu.sync_copy(bufA0, out_hbm.at[pl.ds(ci * CH, CH)])
            return 0

        lax.fori_loop(0, nz, ochunk, 0)

        @pl.when(s == 0)
        def _():
            pltpu.sync_copy(agg_sh.at[pl.ds(NROWF * CH, ROWREM)],
                            bufA0.at[pl.ds(0, ROWREM)])
            pltpu.sync_copy(bufA0.at[pl.ds(0, ROWREM)],
                            out_hbm.at[pl.ds(NROWF * CH, ROWREM)])

    @pl.when(c == 0)
    def _():
        pipeline(a0_hbm, b0_hbm, out0_hbm)

    @pl.when(c == 1)
    def _():
        pipeline(a1_hbm, b1_hbm, out1_hbm)


def _message(a0, a1, b0, b1, srcp, dstp):
    mesh = plsc.VectorSubcoreMesh(core_axis_name="c", subcore_axis_name="s")
    half = jax.ShapeDtypeStruct((N_NODES, HALF), jnp.float32)
    f = pl.kernel(
        _mp_body,
        out_type=[half, half],
        mesh=mesh,
        compiler_params=pltpu.CompilerParams(use_tc_tiling_on_sc=False),
        scratch_types=[
            pltpu.VMEM((CPB, CH), jnp.int32),
            pltpu.VMEM((CPB, CH), jnp.int32),
            pltpu.VMEM((CPB, CH), jnp.int32),
            pltpu.VMEM((CPB, CH), jnp.int32),
            pltpu.VMEM((CH, HALF), jnp.float32),
            pltpu.VMEM((CH, HALF), jnp.float32),
            pltpu.VMEM((CH, HALF), jnp.float32),
            pltpu.VMEM((CH, HALF), jnp.float32),
            pltpu.VMEM((CH, HALF), jnp.float32),
            pltpu.VMEM((CH, HALF), jnp.float32),
            pltpu.VMEM_SHARED((N_NODES, HALF), jnp.float32),
            pltpu.SemaphoreType.DMA,
            pltpu.SemaphoreType.DMA,
            pltpu.SemaphoreType.DMA,
            pltpu.SemaphoreType.DMA,
            pltpu.SemaphoreType.DMA,
        ],
    )
    return f(a0, a1, b0, b1, srcp, dstp)


def kernel(u0, edge_index, ts, W_enc, b_enc, We, be, Wn, bn, W_dec, b_dec):
    ei = edge_index.astype(jnp.int32)
    pad = jnp.zeros((PAD_E - N_EDGES,), jnp.int32)
    srcp = jnp.concatenate([ei[0], pad]).reshape(IDX_ROWS, CH)
    dstp = jnp.concatenate([ei[1], pad]).reshape(IDX_ROWS, CH)
    b_enc2 = b_enc.reshape(1, D_LAT)
    b_dec2 = b_dec.reshape(1, D_IN)

    y = u0
    ys = [y]
    for k in range(T_STEPS - 1):
        dt = (ts[k + 1] - ts[k]).reshape(1, 1)
        h = _encode(y, W_enc, b_enc2)
        for l in range(N_LAYERS):
            a0, a1, bb0, bb1 = _project(h, We[l][:D_LAT], We[l][D_LAT:],
                                        be[l].reshape(1, D_LAT))
            g0, g1 = _message(a0, a1, bb0, bb1, srcp, dstp)
            h = _node_update(h, g0, g1, Wn[l], bn[l].reshape(1, D_LAT))
        y = _decode(h, y, W_dec, b_dec2, dt)
        ys.append(y)
    return jnp.stack(ys, axis=0)


# speedup vs baseline: 8.1849x; 1.4103x over previous
import functools

import jax
import jax.numpy as jnp
from jax import lax
from jax.experimental import pallas as pl
from jax.experimental.pallas import tpu as pltpu
from jax.experimental.pallas import tpu_sc as plsc

N_NODES = 50000
N_EDGES = 800000
D_IN = 128
D_LAT = 64
HALF = D_LAT // 2
N_LAYERS = 4
T_STEPS = 3

RB = 5000
GRID = N_NODES // RB

NS = 16
CH = 128
CPB = 8
BPT = 50
CPT = BPT * CPB
PAD_E = NS * CPT * CH
IDX_ROWS = PAD_E // CH
NROWF = N_NODES // CH
ROWREM = N_NODES - NROWF * CH

_PERM = tuple(range(0, HALF, 2)) + tuple(range(1, HALF, 2))


def _enc_body(u_ref, w_ref, b_ref, h_ref):
    h_ref[...] = jnp.maximum(
        jnp.dot(u_ref[...], w_ref[...], preferred_element_type=jnp.float32)
        + b_ref[...], 0.0)


def _encode(u, W_enc, b_enc2):
    return pl.pallas_call(
        _enc_body,
        grid=(GRID,),
        in_specs=[
            pl.BlockSpec((RB, D_IN), lambda i: (i, 0)),
            pl.BlockSpec((D_IN, D_LAT), lambda i: (0, 0)),
            pl.BlockSpec((1, D_LAT), lambda i: (0, 0)),
        ],
        out_specs=pl.BlockSpec((RB, D_LAT), lambda i: (i, 0)),
        out_shape=jax.ShapeDtypeStruct((N_NODES, D_LAT), jnp.float32),
    )(u, W_enc, b_enc2)


def _proj_body(h_ref, wt_ref, wb_ref, be_ref, a0_ref, a1_ref, b0_ref, b1_ref):
    h = h_ref[...]
    A = jnp.dot(h, wt_ref[...], preferred_element_type=jnp.float32)
    B = jnp.dot(h, wb_ref[...], preferred_element_type=jnp.float32) + be_ref[...]
    a0_ref[...] = A[:, :HALF].astype(jnp.bfloat16)
    a1_ref[...] = A[:, HALF:].astype(jnp.bfloat16)
    b0_ref[...] = B[:, :HALF].astype(jnp.bfloat16)
    b1_ref[...] = B[:, HALF:].astype(jnp.bfloat16)


def _project(h, We_t, We_b, be2):
    half = jax.ShapeDtypeStruct((N_NODES, HALF), jnp.bfloat16)
    return pl.pallas_call(
        _proj_body,
        grid=(GRID,),
        in_specs=[
            pl.BlockSpec((RB, D_LAT), lambda i: (i, 0)),
            pl.BlockSpec((D_LAT, D_LAT), lambda i: (0, 0)),
            pl.BlockSpec((D_LAT, D_LAT), lambda i: (0, 0)),
            pl.BlockSpec((1, D_LAT), lambda i: (0, 0)),
        ],
        out_specs=[pl.BlockSpec((RB, HALF), lambda i: (i, 0))] * 4,
        out_shape=[half, half, half, half],
    )(h, We_t, We_b, be2)


def _node_body(h_ref, g0_ref, g1_ref, w_ref, b_ref, o_ref):
    h = h_ref[...]
    w = w_ref[...]
    acc = jnp.dot(h, w[:D_LAT], preferred_element_type=jnp.float32)
    acc = acc + jnp.dot(g0_ref[...], w[D_LAT:D_LAT + HALF],
                        preferred_element_type=jnp.float32)
    acc = acc + jnp.dot(g1_ref[...], w[D_LAT + HALF:],
                        preferred_element_type=jnp.float32)
    o_ref[...] = h + jnp.maximum(acc + b_ref[...], 0.0)


def _node_update(h, g0, g1, Wn_l, bn2):
    return pl.pallas_call(
        _node_body,
        grid=(GRID,),
        in_specs=[
            pl.BlockSpec((RB, D_LAT), lambda i: (i, 0)),
            pl.BlockSpec((RB, HALF), lambda i: (i, 0)),
            pl.BlockSpec((RB, HALF), lambda i: (i, 0)),
            pl.BlockSpec((2 * D_LAT, D_LAT), lambda i: (0, 0)),
            pl.BlockSpec((1, D_LAT), lambda i: (0, 0)),
        ],
        out_specs=pl.BlockSpec((RB, D_LAT), lambda i: (i, 0)),
        out_shape=jax.ShapeDtypeStruct((N_NODES, D_LAT), jnp.float32),
    )(h, g0, g1, Wn_l, bn2)


def _dec_body(h_ref, y_ref, w_ref, b_ref, dt_ref, o_ref):
    dy = jnp.dot(h_ref[...], w_ref[...],
                 preferred_element_type=jnp.float32) + b_ref[...]
    o_ref[...] = y_ref[...] + dt_ref[...] * dy


def _decode(h, y, W_dec, b_dec2, dt):
    return pl.pallas_call(
        _dec_body,
        grid=(GRID,),
        in_specs=[
            pl.BlockSpec((RB, D_LAT), lambda i: (i, 0)),
            pl.BlockSpec((RB, D_IN), lambda i: (i, 0)),
            pl.BlockSpec((D_LAT, D_IN), lambda i: (0, 0)),
            pl.BlockSpec((1, D_IN), lambda i: (0, 0)),
            pl.BlockSpec((1, 1), lambda i: (0, 0)),
        ],
        out_specs=pl.BlockSpec((RB, D_IN), lambda i: (i, 0)),
        out_shape=jax.ShapeDtypeStruct((N_NODES, D_IN), jnp.float32),
    )(h, y, W_dec, b_dec2, dt)


def _mp_body(a0_hbm, a1_hbm, b0_hbm, b1_hbm, src_hbm, dst_hbm,
             out0_hbm, out1_hbm,
             is0, is1, id0, id1,
             bufA0, bufB0, bufA1, bufB1, bufA2, bufB2, bufA3, bufB3,
             m0, m1, agg_sh,
             isem, gsem0, gsem1, gsem2, gsem3, ssem0, ssem1):
    c = lax.axis_index("c")
    s = lax.axis_index("s")

    ISL = (is0, is1)
    IDL = (id0, id1)
    BA = (bufA0, bufA1, bufA2, bufA3)
    BB = (bufB0, bufB1, bufB2, bufB3)
    MM = (m0, m1)
    GS = (gsem0, gsem1, gsem2, gsem3)
    SS = (ssem0, ssem1)

    zero16 = jnp.zeros((16,), jnp.float32)

    def zero_rows(ref, lo, hi):
        def zr(e, _):
            ref[e, 0:16] = zero16
            ref[e, 16:32] = zero16
            return 0
        lax.fori_loop(lo, hi, zr, 0)

    zero_rows(m0, 0, CH)
    nz = (NROWF + NS - 1) // NS

    def zchunk(k, _):
        ci = s + NS * k

        @pl.when(ci < NROWF)
        def _():
            pltpu.sync_copy(m0, agg_sh.at[pl.ds(ci * CH, CH)])
        return 0

    lax.fori_loop(0, nz, zchunk, 0)

    @pl.when(s == 0)
    def _():
        pltpu.sync_copy(m0.at[pl.ds(0, ROWREM)],
                        agg_sh.at[pl.ds(NROWF * CH, ROWREM)])

    plsc.subcore_barrier()

    rowbase = s * CPT

    def pipeline(a_hbm, b_hbm, out_hbm):
        def issue(p, isl, idl, j):
            pltpu.async_copy(a_hbm.at[isl.at[j]], BA[p], GS[p])
            pltpu.async_copy(b_hbm.at[idl.at[j]], BB[p], GS[p])

        def drain_scatter(p):
            pltpu.make_async_copy(m0, agg_sh.at[pl.ds(0, CH)], SS[p]).wait()

        def process_chunk(g, p, idl, j, off):
            pltpu.make_async_copy(a_hbm.at[pl.ds(0, CH)], BA[g], GS[g]).wait()
            pltpu.make_async_copy(b_hbm.at[pl.ds(0, CH)], BB[g], GS[g]).wait()
            bA = BA[g]
            bB = BB[g]
            bM = MM[p]
            zb = jnp.zeros((32,), jnp.bfloat16)

            def ew(e, _):
                m32 = jnp.maximum(bA[e, 0:32] + bB[e, 0:32], zb)
                lo, hi = plsc.unpack(m32, format=plsc.PackFormat.INTERLEAVED)
                bM[e, 0:16] = lo
                bM[e, 16:32] = hi
                return 0

            lax.fori_loop(0, CH, ew, 0, unroll=4)
            valid = jnp.minimum(CH, jnp.maximum(0, N_EDGES - off))
            zero_rows(bM, valid, CH)
            pltpu.async_copy(bM, agg_sh.at[idl.at[j]], SS[p], add=True)

        pltpu.sync_copy(src_hbm.at[pl.ds(rowbase, CPB)], is0)
        pltpu.sync_copy(dst_hbm.at[pl.ds(rowbase, CPB)], id0)
        issue(0, is0, id0, 0)
        issue(1, is0, id0, 1)

        def blockpair(b2, _):
            for half_i in range(2):
                bb = 2 * b2 + half_i
                isl, idl = ISL[half_i], IDL[half_i]
                isl2, idl2 = ISL[1 - half_i], IDL[1 - half_i]
                rb = rowbase + bb * CPB

                for j in range(CPB):
                    p = j % 2
                    g = j % 4
                    if j >= 2:
                        drain_scatter(p)
                    else:
                        @pl.when(bb >= 1)
                        def _():
                            drain_scatter(p)
                    if j == 1:
                        @pl.when(bb + 1 < BPT)
                        def _():
                            pltpu.async_copy(src_hbm.at[pl.ds(rb + CPB, CPB)],
                                             isl2, isem)
                            pltpu.async_copy(dst_hbm.at[pl.ds(rb + CPB, CPB)],
                                             idl2, isem)
                    if j < CPB - 2:
                        issue((j + 2) % 4, isl, idl, j + 2)
                    elif j == CPB - 2:
                        @pl.when(bb + 1 < BPT)
                        def _():
                            pltpu.make_async_copy(
                                src_hbm.at[pl.ds(0, CPB)], isl2, isem).wait()
                            pltpu.make_async_copy(
                                dst_hbm.at[pl.ds(0, CPB)], idl2, isem).wait()
                            issue((j + 2) % 4, isl2, idl2, 0)
                    else:
                        @pl.when(bb + 1 < BPT)
                        def _():
                            issue((j + 2) % 4, isl2, idl2, 1)
                    process_chunk(g, p, idl, j, (rb + j) * CH)
            return 0

        lax.fori_loop(0, BPT // 2, blockpair, 0)
        drain_scatter(0)
        drain_scatter(1)

        plsc.subcore_barrier()

        def ochunk(k, _):
            ci = s + NS * k

            @pl.when(ci < NROWF)
            def _():
                pltpu.sync_copy(agg_sh.at[pl.ds(ci * CH, CH)], m0)
                pltpu.sync_copy(m0, out_hbm.at[pl.ds(ci * CH, CH)])
            return 0

        lax.fori_loop(0, nz, ochunk, 0)

        @pl.when(s == 0)
        def _():
            pltpu.sync_copy(agg_sh.at[pl.ds(NROWF * CH, ROWREM)],
                            m0.at[pl.ds(0, ROWREM)])
            pltpu.sync_copy(m0.at[pl.ds(0, ROWREM)],
                            out_hbm.at[pl.ds(NROWF * CH, ROWREM)])

    @pl.when(c == 0)
    def _():
        pipeline(a0_hbm, b0_hbm, out0_hbm)

    @pl.when(c == 1)
    def _():
        pipeline(a1_hbm, b1_hbm, out1_hbm)


def _message(a0, a1, b0, b1, srcp, dstp):
    mesh = plsc.VectorSubcoreMesh(core_axis_name="c", subcore_axis_name="s")
    half = jax.ShapeDtypeStruct((N_NODES, HALF), jnp.float32)
    f = pl.kernel(
        _mp_body,
        out_type=[half, half],
        mesh=mesh,
        compiler_params=pltpu.CompilerParams(use_tc_tiling_on_sc=False,
                                             needs_layout_passes=False),
        scratch_types=[
            pltpu.VMEM((CPB, CH), jnp.int32),
            pltpu.VMEM((CPB, CH), jnp.int32),
            pltpu.VMEM((CPB, CH), jnp.int32),
            pltpu.VMEM((CPB, CH), jnp.int32),
            pltpu.VMEM((CH, HALF), jnp.bfloat16),
            pltpu.VMEM((CH, HALF), jnp.bfloat16),
            pltpu.VMEM((CH, HALF), jnp.bfloat16),
            pltpu.VMEM((CH, HALF), jnp.bfloat16),
            pltpu.VMEM((CH, HALF), jnp.bfloat16),
            pltpu.VMEM((CH, HALF), jnp.bfloat16),
            pltpu.VMEM((CH, HALF), jnp.bfloat16),
            pltpu.VMEM((CH, HALF), jnp.bfloat16),
            pltpu.VMEM((CH, HALF), jnp.float32),
            pltpu.VMEM((CH, HALF), jnp.float32),
            pltpu.VMEM_SHARED((N_NODES, HALF), jnp.float32),
            pltpu.SemaphoreType.DMA,
            pltpu.SemaphoreType.DMA,
            pltpu.SemaphoreType.DMA,
            pltpu.SemaphoreType.DMA,
            pltpu.SemaphoreType.DMA,
            pltpu.SemaphoreType.DMA,
            pltpu.SemaphoreType.DMA,
        ],
    )
    return f(a0, a1, b0, b1, srcp, dstp)


def kernel(u0, edge_index, ts, W_enc, b_enc, We, be, Wn, bn, W_dec, b_dec):
    ei = edge_index.astype(jnp.int32)
    pad = jnp.zeros((PAD_E - N_EDGES,), jnp.int32)
    srcp = jnp.concatenate([ei[0], pad]).reshape(IDX_ROWS, CH)
    dstp = jnp.concatenate([ei[1], pad]).reshape(IDX_ROWS, CH)
    b_enc2 = b_enc.reshape(1, D_LAT)
    b_dec2 = b_dec.reshape(1, D_IN)

    y = u0
    ys = [y]
    for k in range(T_STEPS - 1):
        dt = (ts[k + 1] - ts[k]).reshape(1, 1)
        h = _encode(y, W_enc, b_enc2)
        for l in range(N_LAYERS):
            a0, a1, bb0, bb1 = _project(h, We[l][:D_LAT], We[l][D_LAT:],
                                        be[l].reshape(1, D_LAT))
            g0, g1 = _message(a0, a1, bb0, bb1, srcp, dstp)
            Wn_l = Wn[l]
            Wn_adj = jnp.concatenate(
                [Wn_l[:D_LAT],
                 Wn_l[D_LAT:D_LAT + HALF][_PERM, :],
                 Wn_l[D_LAT + HALF:][_PERM, :]], axis=0)
            h = _node_update(h, g0, g1, Wn_adj, bn[l].reshape(1, D_LAT))
        y = _decode(h, y, W_dec, b_dec2, dt)
        ys.append(y)
    return jnp.stack(ys, axis=0)


# speedup vs baseline: 8.4683x; 1.0346x over previous
import functools

import jax
import jax.numpy as jnp
from jax import lax
from jax.experimental import pallas as pl
from jax.experimental.pallas import tpu as pltpu
from jax.experimental.pallas import tpu_sc as plsc

N_NODES = 50000
N_EDGES = 800000
D_IN = 128
D_LAT = 64
HALF = D_LAT // 2
N_LAYERS = 4
T_STEPS = 3

RB = 5000
GRID = N_NODES // RB

NS = 16
CH = 128
CPB = 8
BPT = 50
CPT = BPT * CPB
PAD_E = NS * CPT * CH
IDX_ROWS = PAD_E // CH
NROWF = N_NODES // CH
ROWREM = N_NODES - NROWF * CH

_PERM = tuple(range(0, HALF, 2)) + tuple(range(1, HALF, 2))


def _emit_proj(h, wt_ref, wb_ref, bee_ref, a0_ref, a1_ref, b0_ref, b1_ref):
    A = jnp.dot(h, wt_ref[...], preferred_element_type=jnp.float32)
    B = (jnp.dot(h, wb_ref[...], preferred_element_type=jnp.float32)
         + bee_ref[...])
    a0_ref[...] = A[:, :HALF].astype(jnp.bfloat16)
    a1_ref[...] = A[:, HALF:].astype(jnp.bfloat16)
    b0_ref[...] = B[:, :HALF].astype(jnp.bfloat16)
    b1_ref[...] = B[:, HALF:].astype(jnp.bfloat16)


_W_SPEC = pl.BlockSpec((D_LAT, D_LAT), lambda i: (0, 0))
_BE_SPEC = pl.BlockSpec((1, D_LAT), lambda i: (0, 0))
_H_SPEC = pl.BlockSpec((RB, D_LAT), lambda i: (i, 0))
_G_SPEC = pl.BlockSpec((RB, HALF), lambda i: (i, 0))
_U_SPEC = pl.BlockSpec((RB, D_IN), lambda i: (i, 0))
_HALF_BF16 = jax.ShapeDtypeStruct((N_NODES, HALF), jnp.bfloat16)
_H_F32 = jax.ShapeDtypeStruct((N_NODES, D_LAT), jnp.float32)
_U_F32 = jax.ShapeDtypeStruct((N_NODES, D_IN), jnp.float32)
_PROJ_OUT_SPECS = [_G_SPEC] * 4
_PROJ_OUT_SHAPES = [_HALF_BF16] * 4


def _encproj_body(u_ref, w_ref, b_ref, wt_ref, wb_ref, bee_ref,
                  h_ref, a0_ref, a1_ref, b0_ref, b1_ref):
    h = jnp.maximum(
        jnp.dot(u_ref[...], w_ref[...], preferred_element_type=jnp.float32)
        + b_ref[...], 0.0)
    h_ref[...] = h
    _emit_proj(h, wt_ref, wb_ref, bee_ref, a0_ref, a1_ref, b0_ref, b1_ref)


def _encode_project(u, W_enc, b_enc2, We_t, We_b, be2):
    return pl.pallas_call(
        _encproj_body,
        grid=(GRID,),
        in_specs=[
            _U_SPEC,
            pl.BlockSpec((D_IN, D_LAT), lambda i: (0, 0)),
            _BE_SPEC, _W_SPEC, _W_SPEC, _BE_SPEC,
        ],
        out_specs=[_H_SPEC] + _PROJ_OUT_SPECS,
        out_shape=[_H_F32] + _PROJ_OUT_SHAPES,
    )(u, W_enc, b_enc2, We_t, We_b, be2)


def _node_core(h_ref, g0_ref, g1_ref, w_ref, b_ref):
    h = h_ref[...]
    w = w_ref[...]
    acc = jnp.dot(h, w[:D_LAT], preferred_element_type=jnp.float32)
    acc = acc + jnp.dot(g0_ref[...], w[D_LAT:D_LAT + HALF],
                        preferred_element_type=jnp.float32)
    acc = acc + jnp.dot(g1_ref[...], w[D_LAT + HALF:],
                        preferred_element_type=jnp.float32)
    return h + jnp.maximum(acc + b_ref[...], 0.0)


def _nodeproj_body(h_ref, g0_ref, g1_ref, w_ref, b_ref,
                   wt_ref, wb_ref, bee_ref,
                   o_ref, a0_ref, a1_ref, b0_ref, b1_ref):
    hn = _node_core(h_ref, g0_ref, g1_ref, w_ref, b_ref)
    o_ref[...] = hn
    _emit_proj(hn, wt_ref, wb_ref, bee_ref, a0_ref, a1_ref, b0_ref, b1_ref)


def _node_update_project(h, g0, g1, Wn_l, bn2, We_t, We_b, be2):
    return pl.pallas_call(
        _nodeproj_body,
        grid=(GRID,),
        in_specs=[
            _H_SPEC, _G_SPEC, _G_SPEC,
            pl.BlockSpec((2 * D_LAT, D_LAT), lambda i: (0, 0)),
            _BE_SPEC, _W_SPEC, _W_SPEC, _BE_SPEC,
        ],
        out_specs=[_H_SPEC] + _PROJ_OUT_SPECS,
        out_shape=[_H_F32] + _PROJ_OUT_SHAPES,
    )(h, g0, g1, Wn_l, bn2, We_t, We_b, be2)


def _node_body(h_ref, g0_ref, g1_ref, w_ref, b_ref, o_ref):
    o_ref[...] = _node_core(h_ref, g0_ref, g1_ref, w_ref, b_ref)


def _node_update(h, g0, g1, Wn_l, bn2):
    return pl.pallas_call(
        _node_body,
        grid=(GRID,),
        in_specs=[
            _H_SPEC, _G_SPEC, _G_SPEC,
            pl.BlockSpec((2 * D_LAT, D_LAT), lambda i: (0, 0)),
            _BE_SPEC,
        ],
        out_specs=_H_SPEC,
        out_shape=_H_F32,
    )(h, g0, g1, Wn_l, bn2)


def _decencproj_body(h_ref, y_ref, wd_ref, bd_ref, dt_ref,
                     we_ref, be_ref, wt_ref, wb_ref, bee_ref,
                     y_out_ref, h_ref_out, a0_ref, a1_ref, b0_ref, b1_ref):
    dy = jnp.dot(h_ref[...], wd_ref[...],
                 preferred_element_type=jnp.float32) + bd_ref[...]
    yn = y_ref[...] + dt_ref[...] * dy
    y_out_ref[...] = yn
    h2 = jnp.maximum(
        jnp.dot(yn, we_ref[...], preferred_element_type=jnp.float32)
        + be_ref[...], 0.0)
    h_ref_out[...] = h2
    _emit_proj(h2, wt_ref, wb_ref, bee_ref, a0_ref, a1_ref, b0_ref, b1_ref)


def _decode_encode_project(h, y, W_dec, b_dec2, dt, W_enc, b_enc2,
                           We_t, We_b, be2):
    return pl.pallas_call(
        _decencproj_body,
        grid=(GRID,),
        in_specs=[
            _H_SPEC, _U_SPEC,
            pl.BlockSpec((D_LAT, D_IN), lambda i: (0, 0)),
            pl.BlockSpec((1, D_IN), lambda i: (0, 0)),
            pl.BlockSpec((1, 1), lambda i: (0, 0)),
            pl.BlockSpec((D_IN, D_LAT), lambda i: (0, 0)),
            _BE_SPEC, _W_SPEC, _W_SPEC, _BE_SPEC,
        ],
        out_specs=[_U_SPEC, _H_SPEC] + _PROJ_OUT_SPECS,
        out_shape=[_U_F32, _H_F32] + _PROJ_OUT_SHAPES,
    )(h, y, W_dec, b_dec2, dt, W_enc, b_enc2, We_t, We_b, be2)


def _dec_body(h_ref, y_ref, w_ref, b_ref, dt_ref, o_ref):
    dy = jnp.dot(h_ref[...], w_ref[...],
                 preferred_element_type=jnp.float32) + b_ref[...]
    o_ref[...] = y_ref[...] + dt_ref[...] * dy


def _decode(h, y, W_dec, b_dec2, dt):
    return pl.pallas_call(
        _dec_body,
        grid=(GRID,),
        in_specs=[
            _H_SPEC, _U_SPEC,
            pl.BlockSpec((D_LAT, D_IN), lambda i: (0, 0)),
            pl.BlockSpec((1, D_IN), lambda i: (0, 0)),
            pl.BlockSpec((1, 1), lambda i: (0, 0)),
        ],
        out_specs=_U_SPEC,
        out_shape=_U_F32,
    )(h, y, W_dec, b_dec2, dt)


def _mp_body(a0_hbm, a1_hbm, b0_hbm, b1_hbm, src_hbm, dst_hbm,
             out0_hbm, out1_hbm,
             is0, is1, id0, id1,
             bufA0, bufB0, bufA1, bufB1, bufA2, bufB2, bufA3, bufB3,
             m0, m1, agg_sh,
             isem, gsem0, gsem1, gsem2, gsem3, ssem0, ssem1):
    c = lax.axis_index("c")
    s = lax.axis_index("s")

    ISL = (is0, is1)
    IDL = (id0, id1)
    BA = (bufA0, bufA1, bufA2, bufA3)
    BB = (bufB0, bufB1, bufB2, bufB3)
    MM = (m0, m1)
    GS = (gsem0, gsem1, gsem2, gsem3)
    SS = (ssem0, ssem1)

    zero16 = jnp.zeros((16,), jnp.float32)

    def zero_rows(ref, lo, hi):
        def zr(e, _):
            ref[e, 0:16] = zero16
            ref[e, 16:32] = zero16
            return 0
        lax.fori_loop(lo, hi, zr, 0)

    zero_rows(m0, 0, CH)
    nz = (NROWF + NS - 1) // NS

    def zchunk(k, _):
        ci = s + NS * k

        @pl.when(ci < NROWF)
        def _():
            pltpu.sync_copy(m0, agg_sh.at[pl.ds(ci * CH, CH)])
        return 0

    lax.fori_loop(0, nz, zchunk, 0)

    @pl.when(s == 0)
    def _():
        pltpu.sync_copy(m0.at[pl.ds(0, ROWREM)],
                        agg_sh.at[pl.ds(NROWF * CH, ROWREM)])

    plsc.subcore_barrier()

    rowbase = s * CPT

    def pipeline(a_hbm, b_hbm, out_hbm):
        def issue(p, isl, idl, j):
            pltpu.async_copy(a_hbm.at[isl.at[j]], BA[p], GS[p])
            pltpu.async_copy(b_hbm.at[idl.at[j]], BB[p], GS[p])

        def drain_scatter(p):
            pltpu.make_async_copy(m0, agg_sh.at[pl.ds(0, CH)], SS[p]).wait()

        def process_chunk(g, p, idl, j, off):
            pltpu.make_async_copy(a_hbm.at[pl.ds(0, CH)], BA[g], GS[g]).wait()
            pltpu.make_async_copy(b_hbm.at[pl.ds(0, CH)], BB[g], GS[g]).wait()
            bA = BA[g]
            bB = BB[g]
            bM = MM[p]
            zb = jnp.zeros((32,), jnp.bfloat16)

            def ew(e, _):
                m32 = jnp.maximum(bA[e, 0:32] + bB[e, 0:32], zb)
                lo, hi = plsc.unpack(m32, format=plsc.PackFormat.INTERLEAVED)
                bM[e, 0:16] = lo
                bM[e, 16:32] = hi
                return 0

            lax.fori_loop(0, CH, ew, 0, unroll=4)
            valid = jnp.minimum(CH, jnp.maximum(0, N_EDGES - off))
            zero_rows(bM, valid, CH)
            pltpu.async_copy(bM, agg_sh.at[idl.at[j]], SS[p], add=True)

        pltpu.sync_copy(src_hbm.at[pl.ds(rowbase, CPB)], is0)
        pltpu.sync_copy(dst_hbm.at[pl.ds(rowbase, CPB)], id0)
        issue(0, is0, id0, 0)
        issue(1, is0, id0, 1)

        def blockpair(b2, _):
            for half_i in range(2):
                bb = 2 * b2 + half_i
                isl, idl = ISL[half_i], IDL[half_i]
                isl2, idl2 = ISL[1 - half_i], IDL[1 - half_i]
                rb = rowbase + bb * CPB

                for j in range(CPB):
                    p = j % 2
                    g = j % 4
                    if j >= 2:
                        drain_scatter(p)
                    else:
                        @pl.when(bb >= 1)
                        def _():
                            drain_scatter(p)
                    if j == 1:
                        @pl.when(bb + 1 < BPT)
                        def _():
                            pltpu.async_copy(src_hbm.at[pl.ds(rb + CPB, CPB)],
                                             isl2, isem)
                            pltpu.async_copy(dst_hbm.at[pl.ds(rb + CPB, CPB)],
                                             idl2, isem)
                    if j < CPB - 2:
                        issue((j + 2) % 4, isl, idl, j + 2)
                    elif j == CPB - 2:
                        @pl.when(bb + 1 < BPT)
                        def _():
                            pltpu.make_async_copy(
                                src_hbm.at[pl.ds(0, CPB)], isl2, isem).wait()
                            pltpu.make_async_copy(
                                dst_hbm.at[pl.ds(0, CPB)], idl2, isem).wait()
                            issue((j + 2) % 4, isl2, idl2, 0)
                    else:
                        @pl.when(bb + 1 < BPT)
                        def _():
                            issue((j + 2) % 4, isl2, idl2, 1)
                    process_chunk(g, p, idl, j, (rb + j) * CH)
            return 0

        lax.fori_loop(0, BPT // 2, blockpair, 0)
        drain_scatter(0)
        drain_scatter(1)

        plsc.subcore_barrier()

        def ochunk(k, _):
            ci = s + NS * k

            @pl.when(ci < NROWF)
            def _():
                pltpu.sync_copy(agg_sh.at[pl.ds(ci * CH, CH)], m0)
                pltpu.sync_copy(m0, out_hbm.at[pl.ds(ci * CH, CH)])
            return 0

        lax.fori_loop(0, nz, ochunk, 0)

        @pl.when(s == 0)
        def _():
            pltpu.sync_copy(agg_sh.at[pl.ds(NROWF * CH, ROWREM)],
                            m0.at[pl.ds(0, ROWREM)])
            pltpu.sync_copy(m0.at[pl.ds(0, ROWREM)],
                            out_hbm.at[pl.ds(NROWF * CH, ROWREM)])

    @pl.when(c == 0)
    def _():
        pipeline(a0_hbm, b0_hbm, out0_hbm)

    @pl.when(c == 1)
    def _():
        pipeline(a1_hbm, b1_hbm, out1_hbm)


def _message(a0, a1, b0, b1, srcp, dstp):
    mesh = plsc.VectorSubcoreMesh(core_axis_name="c", subcore_axis_name="s")
    half = jax.ShapeDtypeStruct((N_NODES, HALF), jnp.float32)
    f = pl.kernel(
        _mp_body,
        out_type=[half, half],
        mesh=mesh,
        compiler_params=pltpu.CompilerParams(use_tc_tiling_on_sc=False,
                                             needs_layout_passes=False),
        scratch_types=[
            pltpu.VMEM((CPB, CH), jnp.int32),
            pltpu.VMEM((CPB, CH), jnp.int32),
            pltpu.VMEM((CPB, CH), jnp.int32),
            pltpu.VMEM((CPB, CH), jnp.int32),
            pltpu.VMEM((CH, HALF), jnp.bfloat16),
            pltpu.VMEM((CH, HALF), jnp.bfloat16),
            pltpu.VMEM((CH, HALF), jnp.bfloat16),
            pltpu.VMEM((CH, HALF), jnp.bfloat16),
            pltpu.VMEM((CH, HALF), jnp.bfloat16),
            pltpu.VMEM((CH, HALF), jnp.bfloat16),
            pltpu.VMEM((CH, HALF), jnp.bfloat16),
            pltpu.VMEM((CH, HALF), jnp.bfloat16),
            pltpu.VMEM((CH, HALF), jnp.float32),
            pltpu.VMEM((CH, HALF), jnp.float32),
            pltpu.VMEM_SHARED((N_NODES, HALF), jnp.float32),
            pltpu.SemaphoreType.DMA,
            pltpu.SemaphoreType.DMA,
            pltpu.SemaphoreType.DMA,
            pltpu.SemaphoreType.DMA,
            pltpu.SemaphoreType.DMA,
            pltpu.SemaphoreType.DMA,
            pltpu.SemaphoreType.DMA,
        ],
    )
    return f(a0, a1, b0, b1, srcp, dstp)


def kernel(u0, edge_index, ts, W_enc, b_enc, We, be, Wn, bn, W_dec, b_dec):
    ei = edge_index.astype(jnp.int32)
    pad = jnp.zeros((PAD_E - N_EDGES,), jnp.int32)
    srcp = jnp.concatenate([ei[0], pad]).reshape(IDX_ROWS, CH)
    dstp = jnp.concatenate([ei[1], pad]).reshape(IDX_ROWS, CH)
    b_enc2 = b_enc.reshape(1, D_LAT)
    b_dec2 = b_dec.reshape(1, D_IN)
    We_t = [We[l][:D_LAT] for l in range(N_LAYERS)]
    We_b = [We[l][D_LAT:] for l in range(N_LAYERS)]
    be2 = [be[l].reshape(1, D_LAT) for l in range(N_LAYERS)]
    Wn_adj = [jnp.concatenate(
        [Wn[l][:D_LAT],
         Wn[l][D_LAT:D_LAT + HALF][_PERM, :],
         Wn[l][D_LAT + HALF:][_PERM, :]], axis=0) for l in range(N_LAYERS)]
    bn2 = [bn[l].reshape(1, D_LAT) for l in range(N_LAYERS)]

    y = u0
    ys = [y]
    h, a0, a1, bb0, bb1 = _encode_project(u0, W_enc, b_enc2,
                                          We_t[0], We_b[0], be2[0])
    for k in range(T_STEPS - 1):
        for l in range(N_LAYERS):
            g0, g1 = _message(a0, a1, bb0, bb1, srcp, dstp)
            if l < N_LAYERS - 1:
                h, a0, a1, bb0, bb1 = _node_update_project(
                    h, g0, g1, Wn_adj[l], bn2[l],
                    We_t[l + 1], We_b[l + 1], be2[l + 1])
            else:
                h = _node_update(h, g0, g1, Wn_adj[l], bn2[l])
        dt = (ts[k + 1] - ts[k]).reshape(1, 1)
        if k < T_STEPS - 2:
            y, h, a0, a1, bb0, bb1 = _decode_encode_project(
                h, y, W_dec, b_dec2, dt, W_enc, b_enc2,
                We_t[0], We_b[0], be2[0])
        else:
            y = _decode(h, y, W_dec, b_dec2, dt)
        ys.append(y)
    return jnp.stack(ys, axis=0)
